# baseline (speedup 1.0000x reference)
import functools

import jax
import jax.numpy as jnp
import numpy as np
from jax.experimental import pallas as pl

TB = 64
GP = 256


def _lroll(a, k):
    if k == 0:
        return a
    return jnp.concatenate([a[..., k:], a[..., :k]], axis=-1)


def _net_kernel(x_ref, w1s_ref, b1_ref, mask_ref, w2r_ref, sel0_ref, wl_ref,
                bl_ref, out_ref):
    w1s = w1s_ref[...]
    zpad = jnp.zeros((144, GP - 225), jnp.bfloat16)
    ms = []
    for i in range(TB):
        m = jnp.dot(w1s, x_ref[i].astype(jnp.bfloat16),
                    preferred_element_type=jnp.float32).astype(jnp.bfloat16)
        ms += [m, zpad]
    mbig = jnp.concatenate(ms, axis=1)
    acc = jnp.broadcast_to(b1_ref[...], (16, TB * GP))
    for t in range(9):
        dy, dx = t // 3, t % 3
        off = (dy - 1) * 15 + (dx - 1)
        acc = acc + _lroll(mbig[16 * t:16 * (t + 1), :], off) * mask_ref[t:t + 1, :]
    ax = jnp.maximum(jnp.maximum(acc, _lroll(acc, 1)), _lroll(acc, 2))
    ay = jnp.maximum(jnp.maximum(ax, _lroll(ax, 15)), _lroll(ax, 30))
    xrow = jnp.concatenate([_lroll(ay, 2 * dx) for dx in range(5)], axis=0)
    g = jnp.dot(w2r_ref[...], xrow,
                preferred_element_type=jnp.float32).astype(jnp.bfloat16)
    d = g[0:32]
    for dy in range(1, 5):
        d = d + _lroll(g[32 * dy:32 * (dy + 1)], 30 * dy)
    e = jnp.maximum(jnp.maximum(d, _lroll(d, 2)), _lroll(d, 4))
    f = jnp.maximum(jnp.maximum(e, _lroll(e, 30)), _lroll(e, 60))
    fall = jnp.dot(f, sel0_ref[...],
                   preferred_element_type=jnp.float32)
    logits = jnp.dot(wl_ref[...], fall.astype(jnp.bfloat16),
                     preferred_element_type=jnp.float32) + bl_ref[...]
    z = logits - jnp.max(logits, axis=0, keepdims=True)
    ez = jnp.exp(z)
    out_ref[0] = ez / jnp.sum(ez, axis=0, keepdims=True)


@functools.partial(jax.jit, static_argnames=())
def kernel(x, W1, b1, W2, Wl, bl):
    B, C, H, W = x.shape
    S = H * W
    x2 = x.reshape(B, C, S)
    w1s = jnp.transpose(W1, (2, 3, 0, 1)).reshape(9 * 16, C).astype(jnp.bfloat16)
    b1c = b1.reshape(16, 1).astype(jnp.bfloat16)
    pp = np.arange(TB * GP) % GP
    yy, xx = np.divmod(pp, W)
    inside = pp < S
    masks = np.zeros((9, TB * GP), dtype=np.float32)
    for t in range(9):
        dy, dx = t // 3, t % 3
        masks[t] = (inside & (yy + dy - 1 >= 0) & (yy + dy - 1 < H)
                    & (xx + dx - 1 >= 0) & (xx + dx - 1 < W))
    mask9 = jnp.asarray(masks, dtype=jnp.bfloat16)
    w2r = jnp.transpose(W2, (2, 0, 3, 1)).reshape(160, 80).astype(jnp.bfloat16)
    sel0n = np.zeros((TB * GP, TB), dtype=np.float32)
    for i in range(TB):
        sel0n[GP * i, i] = 1.0
    sel0 = jnp.asarray(sel0n, dtype=jnp.bfloat16)
    wlb = Wl.astype(jnp.bfloat16)
    blc = bl.reshape(2, 1)

    grid = (B // TB,)
    out = pl.pallas_call(
        _net_kernel,
        grid=grid,
        in_specs=[
            pl.BlockSpec((TB, C, S), lambda i: (i, 0, 0)),
            pl.BlockSpec((144, C), lambda i: (0, 0)),
            pl.BlockSpec((16, 1), lambda i: (0, 0)),
            pl.BlockSpec((9, TB * GP), lambda i: (0, 0)),
            pl.BlockSpec((160, 80), lambda i: (0, 0)),
            pl.BlockSpec((TB * GP, TB), lambda i: (0, 0)),
            pl.BlockSpec((2, 32), lambda i: (0, 0)),
            pl.BlockSpec((2, 1), lambda i: (0, 0)),
        ],
        out_specs=pl.BlockSpec((1, 2, TB), lambda i: (i, 0, 0)),
        out_shape=jax.ShapeDtypeStruct((B // TB, 2, TB), jnp.float32),
    )(x2, w1s, b1c, mask9, w2r, sel0, wlb, blc)
    return out.transpose(0, 2, 1).reshape(B, 2)


# speedup vs baseline: 1.0023x; 1.0023x over previous
import functools

import jax
import jax.numpy as jnp
import numpy as np
from jax.experimental import pallas as pl

TB = 32
GP = 256


def _lroll(a, k):
    if k == 0:
        return a
    return jnp.concatenate([a[..., k:], a[..., :k]], axis=-1)


def _net_kernel(x_ref, w1s_ref, b1_ref, mask_ref, w2r_ref, sel0_ref, wl_ref,
                bl_ref, out_ref):
    w1s = w1s_ref[...]
    zpad = jnp.zeros((144, GP - 225), jnp.bfloat16)
    ms = []
    for i in range(TB):
        m = jnp.dot(w1s, x_ref[i].astype(jnp.bfloat16),
                    preferred_element_type=jnp.float32).astype(jnp.bfloat16)
        ms += [m, zpad]
    mbig = jnp.concatenate(ms, axis=1)
    acc = jnp.broadcast_to(b1_ref[...], (16, TB * GP))
    for t in range(9):
        dy, dx = t // 3, t % 3
        off = (dy - 1) * 15 + (dx - 1)
        acc = acc + _lroll(mbig[16 * t:16 * (t + 1), :], off) * mask_ref[t:t + 1, :]
    ax = jnp.maximum(jnp.maximum(acc, _lroll(acc, 1)), _lroll(acc, 2))
    ay = jnp.maximum(jnp.maximum(ax, _lroll(ax, 15)), _lroll(ax, 30))
    xrow = jnp.concatenate([_lroll(ay, 2 * dx) for dx in range(5)], axis=0)
    g = jnp.dot(w2r_ref[...], xrow,
                preferred_element_type=jnp.float32).astype(jnp.bfloat16)
    d = g[0:32]
    for dy in range(1, 5):
        d = d + _lroll(g[32 * dy:32 * (dy + 1)], 30 * dy)
    e = jnp.maximum(jnp.maximum(d, _lroll(d, 2)), _lroll(d, 4))
    f = jnp.maximum(jnp.maximum(e, _lroll(e, 30)), _lroll(e, 60))
    fall = jnp.dot(f, sel0_ref[...],
                   preferred_element_type=jnp.float32)
    logits = jnp.dot(wl_ref[...], fall.astype(jnp.bfloat16),
                     preferred_element_type=jnp.float32) + bl_ref[...]
    z = logits - jnp.max(logits, axis=0, keepdims=True)
    ez = jnp.exp(z)
    out_ref[0] = ez / jnp.sum(ez, axis=0, keepdims=True)


@functools.partial(jax.jit, static_argnames=())
def kernel(x, W1, b1, W2, Wl, bl):
    B, C, H, W = x.shape
    S = H * W
    x2 = x.reshape(B, C, S)
    w1s = jnp.transpose(W1, (2, 3, 0, 1)).reshape(9 * 16, C).astype(jnp.bfloat16)
    b1c = b1.reshape(16, 1).astype(jnp.bfloat16)
    pp = np.arange(TB * GP) % GP
    yy, xx = np.divmod(pp, W)
    inside = pp < S
    masks = np.zeros((9, TB * GP), dtype=np.float32)
    for t in range(9):
        dy, dx = t // 3, t % 3
        masks[t] = (inside & (yy + dy - 1 >= 0) & (yy + dy - 1 < H)
                    & (xx + dx - 1 >= 0) & (xx + dx - 1 < W))
    mask9 = jnp.asarray(masks, dtype=jnp.bfloat16)
    w2r = jnp.transpose(W2, (2, 0, 3, 1)).reshape(160, 80).astype(jnp.bfloat16)
    sel0n = np.zeros((TB * GP, TB), dtype=np.float32)
    for i in range(TB):
        sel0n[GP * i, i] = 1.0
    sel0 = jnp.asarray(sel0n, dtype=jnp.bfloat16)
    wlb = Wl.astype(jnp.bfloat16)
    blc = bl.reshape(2, 1)

    grid = (B // TB,)
    out = pl.pallas_call(
        _net_kernel,
        grid=grid,
        in_specs=[
            pl.BlockSpec((TB, C, S), lambda i: (i, 0, 0)),
            pl.BlockSpec((144, C), lambda i: (0, 0)),
            pl.BlockSpec((16, 1), lambda i: (0, 0)),
            pl.BlockSpec((9, TB * GP), lambda i: (0, 0)),
            pl.BlockSpec((160, 80), lambda i: (0, 0)),
            pl.BlockSpec((TB * GP, TB), lambda i: (0, 0)),
            pl.BlockSpec((2, 32), lambda i: (0, 0)),
            pl.BlockSpec((2, 1), lambda i: (0, 0)),
        ],
        out_specs=pl.BlockSpec((1, 2, TB), lambda i: (i, 0, 0)),
        out_shape=jax.ShapeDtypeStruct((B // TB, 2, TB), jnp.float32),
    )(x2, w1s, b1c, mask9, w2r, sel0, wlb, blc)
    return out.transpose(0, 2, 1).reshape(B, 2)
